# baseline (speedup 1.0000x reference)
import functools

import jax
import jax.numpy as jnp
from jax import lax
from jax.experimental import pallas as pl
from jax.experimental.pallas import tpu as pltpu
from jax.experimental.pallas import tpu_sc as plsc

_N_EDGE = 32640
_TILES = 255
_CH_T = 17
_CH_IN = _CH_T * 128
_CH_OUT = _CH_T * 1024
_N_CH = _TILES // _CH_T


def _edge_sc_body(x_hbm, out_hbm, xbuf, obuf):
    wid = lax.axis_index("c") * 16 + lax.axis_index("s")
    neg = jnp.full((16,), -100.0, jnp.float32)
    hun = jnp.full((16,), 100.0, jnp.float32)
    iota = lax.iota(jnp.int32, 16)

    def fill(k, carry):
        obuf[pl.ds(k * 16, 16)] = neg
        return carry

    lax.fori_loop(0, _CH_OUT // 16, fill, 0)

    def scatter_pass(vec):
        def per_tile(t, carry):
            for j in range(8):
                xv = xbuf[pl.ds(t * 128 + j * 16, 16)]
                idx = t * 1024 + j * 16 + xv * 128 + iota
                plsc.store_scatter(obuf, [idx], vec)
            return carry

        lax.fori_loop(0, _CH_T, per_tile, 0)

    def per_batch(bi, carry):
        b = wid * 2 + bi

        def per_chunk(c, carry2):
            pltpu.sync_copy(
                x_hbm.at[pl.ds(b * _N_EDGE + c * _CH_IN, _CH_IN)], xbuf
            )
            scatter_pass(hun)
            pltpu.sync_copy(
                obuf,
                out_hbm.at[pl.ds(b * (_TILES * 1024) + c * _CH_OUT, _CH_OUT)],
            )
            scatter_pass(neg)
            return carry2

        lax.fori_loop(0, _N_CH, per_chunk, 0)
        return carry

    lax.fori_loop(0, 2, per_batch, 0)


def _node_kernel(x_ref, node_ref):
    xf = x_ref[...].astype(jnp.float32)
    lane = jax.lax.broadcasted_iota(jnp.int32, (1, 128), 1).astype(jnp.float32)
    ones = jnp.ones((1, 128), dtype=jnp.float32)
    for k in range(8):
        xn = xf[k : k + 1, :]
        col = jax.lax.dot_general(
            xn,
            ones,
            dimension_numbers=(((0,), (0,)), ((), ())),
            preferred_element_type=jnp.float32,
        )
        node_ref[k] = jnp.where(col == lane, 100.0, -100.0)


def kernel(tokens, pad_mask, t, x0):
    B = x0.shape[0]
    node = pl.pallas_call(
        _node_kernel,
        grid=(B // 8,),
        in_specs=[pl.BlockSpec((8, 256), lambda i: (i, 0))],
        out_specs=pl.BlockSpec((8, 256, 128), lambda i: (i, 0, 0)),
        out_shape=jax.ShapeDtypeStruct((B, 256, 128), jnp.float32),
    )(x0[:, :256])

    x_lin = x0[:, 256:].reshape(-1)
    mesh = plsc.VectorSubcoreMesh(core_axis_name="c", subcore_axis_name="s")
    edge_fn = functools.partial(
        pl.kernel,
        out_type=jax.ShapeDtypeStruct((B * _N_EDGE * 8,), jnp.float32),
        scratch_types=[
            pltpu.VMEM((_CH_IN,), jnp.int32),
            pltpu.VMEM((_CH_OUT,), jnp.float32),
        ],
        mesh=mesh,
        compiler_params=pltpu.CompilerParams(needs_layout_passes=False),
    )(_edge_sc_body)
    edge1d = edge_fn(x_lin)
    edge = (
        edge1d.reshape(B, _TILES, 8, 128)
        .transpose(0, 1, 3, 2)
        .reshape(B, _N_EDGE, 8)
    )
    return node, edge


# speedup vs baseline: 4.9384x; 4.9384x over previous
import jax
import jax.numpy as jnp
from jax.experimental import pallas as pl

_SEQ = 32896
_N_EDGE = 32640
_GB = 8


def _onehot_kernel(x_ref, node_ref, edge_ref):
    xf = x_ref[0].astype(jnp.float32)

    j_row = jax.lax.broadcasted_iota(jnp.int32, (_GB * 8, _GB), 0)
    j_col = jax.lax.broadcasted_iota(jnp.int32, (_GB * 8, _GB), 1)
    rep = (j_col == j_row // 8).astype(jnp.float32)
    r = jax.lax.dot(rep, xf, preferred_element_type=jnp.float32)
    vrow = (
        jax.lax.broadcasted_iota(jnp.int32, (_GB * 8, 1), 0) % 8
    ).astype(jnp.float32)
    edge = jnp.where(r[:, 256:] == vrow, 100.0, -100.0)
    edge_ref[...] = edge.reshape(_GB, 8, _N_EDGE)

    lane = jax.lax.broadcasted_iota(jnp.int32, (1, 128), 1).astype(jnp.float32)
    ones = jnp.ones((1, 128), dtype=jnp.float32)
    for k in range(_GB):
        xn = xf[k : k + 1, :256]
        col = jax.lax.dot_general(
            xn,
            ones,
            dimension_numbers=(((0,), (0,)), ((), ())),
            preferred_element_type=jnp.float32,
        )
        node_ref[k] = jnp.where(col == lane, 100.0, -100.0)


def kernel(tokens, pad_mask, t, x0):
    B = x0.shape[0]
    xr = x0.reshape(B // _GB, _GB, _SEQ)
    node, edge_vm = pl.pallas_call(
        _onehot_kernel,
        grid=(B // _GB,),
        in_specs=[pl.BlockSpec((1, _GB, _SEQ), lambda i: (i, 0, 0))],
        out_specs=[
            pl.BlockSpec((_GB, 256, 128), lambda i: (i, 0, 0)),
            pl.BlockSpec((_GB, 8, _N_EDGE), lambda i: (i, 0, 0)),
        ],
        out_shape=[
            jax.ShapeDtypeStruct((B, 256, 128), jnp.float32),
            jax.ShapeDtypeStruct((B, 8, _N_EDGE), jnp.float32),
        ],
    )(xr)
    return node, edge_vm.transpose(0, 2, 1)


# speedup vs baseline: 4.9391x; 1.0001x over previous
import jax
import jax.numpy as jnp
from jax.experimental import pallas as pl
from jax.experimental.pallas import tpu as pltpu

_SEQ = 32896
_N_EDGE = 32640
_GB = 8


def _onehot_kernel(x_ref, node_ref, edge_ref):
    xf = x_ref[0].astype(jnp.float32)

    j_row = jax.lax.broadcasted_iota(jnp.int32, (_GB * 8, _GB), 0)
    j_col = jax.lax.broadcasted_iota(jnp.int32, (_GB * 8, _GB), 1)
    rep = (j_col == j_row // 8).astype(jnp.float32)
    r = jax.lax.dot(rep, xf, preferred_element_type=jnp.float32)
    vrow = (
        jax.lax.broadcasted_iota(jnp.int32, (_GB * 8, 1), 0) % 8
    ).astype(jnp.float32)
    edge = jnp.where(r[:, 256:] == vrow, 100.0, -100.0)
    edge_ref[...] = edge.reshape(_GB, 8, _N_EDGE)

    lane = jax.lax.broadcasted_iota(jnp.int32, (1, 128), 1).astype(jnp.float32)
    ones = jnp.ones((1, 128), dtype=jnp.float32)
    for k in range(_GB):
        xn = xf[k : k + 1, :256]
        col = jax.lax.dot_general(
            xn,
            ones,
            dimension_numbers=(((0,), (0,)), ((), ())),
            preferred_element_type=jnp.float32,
        )
        node_ref[k] = jnp.where(col == lane, 100.0, -100.0)


def kernel(tokens, pad_mask, t, x0):
    B = x0.shape[0]
    xr = x0.reshape(B // _GB, _GB, _SEQ)
    node, edge_vm = pl.pallas_call(
        _onehot_kernel,
        grid=(B // _GB,),
        in_specs=[pl.BlockSpec((1, _GB, _SEQ), lambda i: (i, 0, 0))],
        out_specs=[
            pl.BlockSpec((_GB, 256, 128), lambda i: (i, 0, 0)),
            pl.BlockSpec((_GB, 8, _N_EDGE), lambda i: (i, 0, 0)),
        ],
        out_shape=[
            jax.ShapeDtypeStruct((B, 256, 128), jnp.float32),
            jax.ShapeDtypeStruct((B, 8, _N_EDGE), jnp.float32),
        ],
        compiler_params=pltpu.CompilerParams(
            dimension_semantics=("parallel",)
        ),
    )(xr)
    return node, edge_vm.transpose(0, 2, 1)
